# baseline (speedup 1.0000x reference)
import functools

import jax
import jax.numpy as jnp
from jax import lax
from jax.experimental import pallas as pl
from jax.experimental.pallas import tpu as pltpu
from jax.experimental.pallas import tpu_sc as plsc

NC = 2
NS = 16
NW = NC * NS
CHUNK = 64
NBUF = 6


@functools.lru_cache(maxsize=None)
def _make_sc_gather(Bs, V, D, t_off):
    b_per_w = Bs // NW
    n_chunks = b_per_w // CHUNK
    nbuf = min(NBUF, n_chunks)
    mesh = plsc.VectorSubcoreMesh(core_axis_name="c", subcore_axis_name="s")

    @functools.partial(
        pl.kernel,
        mesh=mesh,
        out_type=jax.ShapeDtypeStruct((Bs, D), jnp.float32),
        scratch_types=(
            [pltpu.VMEM((b_per_w,), jnp.int32)]
            + [pltpu.VMEM((CHUNK, D), jnp.float32) for _ in range(nbuf)]
            + [pltpu.SemaphoreType.DMA for _ in range(2 * nbuf)]
        ),
    )
    def gather_kernel(idx_hbm, table_hbm, out_hbm, idx_v, *refs):
        rows = list(refs[:nbuf])
        gsem = list(refs[nbuf:2 * nbuf])
        wsem = list(refs[2 * nbuf:3 * nbuf])

        wid = lax.axis_index("s") * NC + lax.axis_index("c")
        base = wid * b_per_w

        pltpu.sync_copy(idx_hbm.at[pl.ds(t_off + base, b_per_w)], idx_v)

        def fire_gather(c):
            return pltpu.async_copy(
                table_hbm.at[idx_v.at[pl.ds(c * CHUNK, CHUNK)]],
                rows[c % nbuf], gsem[c % nbuf])

        gcp = [None] * nbuf
        wcp = [None] * n_chunks
        w_done = [False] * n_chunks
        for c in range(nbuf):
            gcp[c % nbuf] = fire_gather(c)
        for c in range(n_chunks):
            gcp[c % nbuf].wait()
            wcp[c] = pltpu.async_copy(
                rows[c % nbuf],
                out_hbm.at[pl.ds(base + c * CHUNK, CHUNK)],
                wsem[c % nbuf])
            if c + nbuf < n_chunks:
                wcp[c].wait()
                w_done[c] = True
                gcp[c % nbuf] = fire_gather(c + nbuf)
        for c in range(n_chunks):
            if not w_done[c]:
                wcp[c].wait()

    return gather_kernel


@functools.lru_cache(maxsize=None)
def _make_tc_proj(Bs, D, E, blk, out_B, out_off, alias_prev):
    nblk = Bs // blk
    off = out_off // blk

    def body(*refs):
        x_ref, w_ref, b_ref = refs[0], refs[1], refs[2]
        o_ref = refs[-1]
        o_ref[...] = lax.dot_general(
            x_ref[...], w_ref[...],
            (((1,), (1,)), ((), ())),
            preferred_element_type=jnp.float32,
        ) + b_ref[...]

    in_specs = [
        pl.BlockSpec((blk, D), lambda i: (i, 0)),
        pl.BlockSpec((E, D), lambda i: (0, 0)),
        pl.BlockSpec((1, E), lambda i: (0, 0)),
    ]
    if alias_prev:
        in_specs.append(pl.BlockSpec(memory_space=pl.ANY))

    return pl.pallas_call(
        body,
        grid=(nblk,),
        in_specs=in_specs,
        out_specs=pl.BlockSpec((blk, E), lambda i: (i + off, 0)),
        out_shape=jax.ShapeDtypeStruct((out_B, E), jnp.float32),
        input_output_aliases={3: 0} if alias_prev else {},
    )


def kernel(t, pos_enc, W, b):
    B = t.shape[0]
    V, D = pos_enc.shape
    E = W.shape[0]
    B0 = (B * 5) // 8
    B1 = B - B0

    emb0 = _make_sc_gather(B0, V, D, 0)(t, pos_enc)
    emb1 = _make_sc_gather(B1, V, D, B0)(t, pos_enc)
    b2 = b.reshape(1, E)
    out0 = _make_tc_proj(B0, D, E, 2048, B, 0, False)(emb0, W, b2)
    return _make_tc_proj(B1, D, E, 2048, B, B0, True)(emb1, W, b2, out0)


# speedup vs baseline: 1.0759x; 1.0759x over previous
import functools

import jax
import jax.numpy as jnp
from jax import lax
from jax.experimental import pallas as pl
from jax.experimental.pallas import tpu as pltpu
from jax.experimental.pallas import tpu_sc as plsc

NC = 2
NS = 16
NW = NC * NS
CHUNK = 64
NBUF = 2


@functools.lru_cache(maxsize=None)
def _make_sc_gather(Bs, V, D, t_off):
    b_per_w = Bs // NW
    n_chunks = b_per_w // CHUNK
    nbuf = min(NBUF, n_chunks // 2)
    mesh = plsc.VectorSubcoreMesh(core_axis_name="c", subcore_axis_name="s")

    @functools.partial(
        pl.kernel,
        mesh=mesh,
        out_type=jax.ShapeDtypeStruct((Bs, D), jnp.float32),
        scratch_types=(
            [pltpu.VMEM((b_per_w,), jnp.int32)]
            + [pltpu.VMEM((2 * CHUNK, D), jnp.float32) for _ in range(nbuf)]
            + [pltpu.SemaphoreType.DMA for _ in range(2 * nbuf)]
        ),
    )
    def gather_kernel(idx_hbm, table_hbm, out_hbm, idx_v, *refs):
        rows = list(refs[:nbuf])
        gsem = list(refs[nbuf:2 * nbuf])
        wsem = list(refs[2 * nbuf:3 * nbuf])

        wid = lax.axis_index("s") * NC + lax.axis_index("c")
        base = wid * b_per_w

        pltpu.sync_copy(idx_hbm.at[pl.ds(t_off + base, b_per_w)], idx_v)

        def fire_pair(p):
            buf = rows[p % nbuf]
            sem = gsem[p % nbuf]
            return [
                pltpu.async_copy(
                    table_hbm.at[idx_v.at[pl.ds((2 * p + h) * CHUNK, CHUNK)]],
                    buf.at[pl.ds(h * CHUNK, CHUNK)], sem)
                for h in range(2)
            ]

        n_pairs = n_chunks // 2
        gcp = [None] * nbuf
        wcp = [None] * n_pairs
        w_done = [False] * n_pairs
        for p in range(min(nbuf, n_pairs)):
            gcp[p % nbuf] = fire_pair(p)
        for p in range(n_pairs):
            for cp in gcp[p % nbuf]:
                cp.wait()
            wcp[p] = pltpu.async_copy(
                rows[p % nbuf],
                out_hbm.at[pl.ds(base + 2 * p * CHUNK, 2 * CHUNK)],
                wsem[p % nbuf])
            if p + nbuf < n_pairs:
                wcp[p].wait()
                w_done[p] = True
                gcp[p % nbuf] = fire_pair(p + nbuf)
        for p in range(n_pairs):
            if not w_done[p]:
                wcp[p].wait()

    return gather_kernel


@functools.lru_cache(maxsize=None)
def _make_tc_proj(Bs, D, E, blk, out_B, out_off, alias_prev):
    nblk = Bs // blk
    off = out_off // blk

    def body(*refs):
        x_ref, w_ref, b_ref = refs[0], refs[1], refs[2]
        o_ref = refs[-1]
        o_ref[...] = lax.dot_general(
            x_ref[...], w_ref[...],
            (((1,), (1,)), ((), ())),
            preferred_element_type=jnp.float32,
        ) + b_ref[...]

    in_specs = [
        pl.BlockSpec((blk, D), lambda i: (i, 0)),
        pl.BlockSpec((E, D), lambda i: (0, 0)),
        pl.BlockSpec((1, E), lambda i: (0, 0)),
    ]
    if alias_prev:
        in_specs.append(pl.BlockSpec(memory_space=pl.ANY))

    return pl.pallas_call(
        body,
        grid=(nblk,),
        in_specs=in_specs,
        out_specs=pl.BlockSpec((blk, E), lambda i: (i + off, 0)),
        out_shape=jax.ShapeDtypeStruct((out_B, E), jnp.float32),
        input_output_aliases={3: 0} if alias_prev else {},
    )


def kernel(t, pos_enc, W, b):
    B = t.shape[0]
    V, D = pos_enc.shape
    E = W.shape[0]
    embed = _make_sc_gather(B, V, D, 0)(t, pos_enc)
    return _make_tc_proj(B, D, E, 8192, B, 0, False)(embed, W, b.reshape(1, E))


# speedup vs baseline: 1.1254x; 1.0460x over previous
import functools

import jax
import jax.numpy as jnp
from jax import lax
from jax.experimental import pallas as pl
from jax.experimental.pallas import tpu as pltpu
from jax.experimental.pallas import tpu_sc as plsc

NC = 2
NS = 16
NW = NC * NS
CHUNK = 64
NBUF = 6


@functools.lru_cache(maxsize=None)
def _make_sc_gather(Bs, V, D, t_off):
    b_per_w = Bs // NW
    n_chunks = b_per_w // CHUNK
    nbuf = min(NBUF, n_chunks)
    mesh = plsc.VectorSubcoreMesh(core_axis_name="c", subcore_axis_name="s")

    @functools.partial(
        pl.kernel,
        mesh=mesh,
        out_type=jax.ShapeDtypeStruct((Bs, D), jnp.float32),
        scratch_types=(
            [pltpu.VMEM((b_per_w,), jnp.int32)]
            + [pltpu.VMEM((CHUNK, D), jnp.float32) for _ in range(nbuf)]
            + [pltpu.SemaphoreType.DMA for _ in range(2 * nbuf)]
        ),
    )
    def gather_kernel(idx_hbm, table_hbm, out_hbm, idx_v, *refs):
        rows = list(refs[:nbuf])
        gsem = list(refs[nbuf:2 * nbuf])
        wsem = list(refs[2 * nbuf:3 * nbuf])

        wid = lax.axis_index("s") * NC + lax.axis_index("c")
        base = wid * b_per_w

        pltpu.sync_copy(idx_hbm.at[pl.ds(t_off + base, b_per_w)], idx_v)

        def fire_gather(c):
            return pltpu.async_copy(
                table_hbm.at[idx_v.at[pl.ds(c * CHUNK, CHUNK)]],
                rows[c % nbuf], gsem[c % nbuf])

        gcp = [None] * nbuf
        wcp = [None] * n_chunks
        w_done = [False] * n_chunks
        for c in range(nbuf):
            gcp[c % nbuf] = fire_gather(c)
        for c in range(n_chunks):
            gcp[c % nbuf].wait()
            wcp[c] = pltpu.async_copy(
                rows[c % nbuf],
                out_hbm.at[pl.ds(base + c * CHUNK, CHUNK)],
                wsem[c % nbuf])
            if c + nbuf < n_chunks:
                wcp[c].wait()
                w_done[c] = True
                gcp[c % nbuf] = fire_gather(c + nbuf)
        for c in range(n_chunks):
            if not w_done[c]:
                wcp[c].wait()

    return gather_kernel


@functools.lru_cache(maxsize=None)
def _make_tc_proj(Bs, D, E, blk, out_B, out_off, alias_prev):
    nblk = Bs // blk
    off = out_off // blk

    def body(*refs):
        x_ref, w_ref, b_ref = refs[0], refs[1], refs[2]
        o_ref = refs[-1]
        o_ref[...] = lax.dot_general(
            x_ref[...], w_ref[...],
            (((1,), (1,)), ((), ())),
            preferred_element_type=jnp.float32,
        ) + b_ref[...]

    in_specs = [
        pl.BlockSpec((blk, D), lambda i: (i, 0)),
        pl.BlockSpec((E, D), lambda i: (0, 0)),
        pl.BlockSpec((1, E), lambda i: (0, 0)),
    ]
    if alias_prev:
        in_specs.append(pl.BlockSpec(memory_space=pl.ANY))

    return pl.pallas_call(
        body,
        grid=(nblk,),
        in_specs=in_specs,
        out_specs=pl.BlockSpec((blk, E), lambda i: (i + off, 0)),
        out_shape=jax.ShapeDtypeStruct((out_B, E), jnp.float32),
        input_output_aliases={3: 0} if alias_prev else {},
    )


def kernel(t, pos_enc, W, b):
    B = t.shape[0]
    V, D = pos_enc.shape
    E = W.shape[0]
    embed = _make_sc_gather(B, V, D, 0)(t, pos_enc)
    return _make_tc_proj(B, D, E, 8192, B, 0, False)(embed, W, b.reshape(1, E))
